# baseline (speedup 1.0000x reference)
import functools

import jax
import jax.numpy as jnp
from jax import lax
from jax.experimental import pallas as pl
from jax.experimental.pallas import tpu as pltpu
from jax.experimental.pallas import tpu_sc as plsc

_B, _T, _N_IN, _N_OUT, _K = 8, 1024, 96, 32, 8192
_BT = _B * _T

_TM = 256
_KC = 2048


def _argmin_body(x_ref, w_ref, b_ref, cb_ref, idx_ref):
    xt = x_ref[...]
    wt = w_ref[...]
    out = lax.dot_general(xt, wt, (((1,), (1,)), ((), ())),
                          preferred_element_type=jnp.float32)
    out = out + b_ref[0, :][None, :]
    out2 = jnp.sum(out * out, axis=1, keepdims=True)

    def step(c, carry):
        best_d, best_i = carry
        cbc = cb_ref[pl.ds(c * _KC, _KC), :]
        s = lax.dot_general(out, cbc, (((1,), (1,)), ((), ())),
                            preferred_element_type=jnp.float32)
        c2 = jnp.sum(cbc * cbc, axis=1)[None, :]
        d = out2 - 2.0 * s + c2
        md = jnp.min(d, axis=1, keepdims=True)
        ii = lax.broadcasted_iota(jnp.int32, d.shape, 1) + c * _KC
        mi = jnp.min(jnp.where(d == md, ii, jnp.int32(_K)), axis=1,
                     keepdims=True)
        take = md < best_d
        return (jnp.where(take, md, best_d), jnp.where(take, mi, best_i))

    init = (jnp.full((_TM, 1), jnp.inf, jnp.float32),
            jnp.zeros((_TM, 1), jnp.int32))
    _, best_i = lax.fori_loop(0, _K // _KC, step, init, unroll=True)
    idx_ref[...] = best_i[:, 0].reshape(1, 1, _TM)


def _nearest_idx(x2d, w, b2d, cb):
    grid = (_BT // _TM,)
    return pl.pallas_call(
        _argmin_body,
        grid=grid,
        in_specs=[
            pl.BlockSpec((_TM, _N_IN), lambda i: (i, 0)),
            pl.BlockSpec((_N_OUT, _N_IN), lambda i: (0, 0)),
            pl.BlockSpec((1, _N_OUT), lambda i: (0, 0)),
            pl.BlockSpec((_K, _N_OUT), lambda i: (0, 0)),
        ],
        out_specs=pl.BlockSpec((1, 1, _TM), lambda i: (i, 0, 0)),
        out_shape=jax.ShapeDtypeStruct((_BT // _TM, 1, _TM), jnp.int32),
    )(x2d, w, b2d, cb)


_NC, _NS = 2, 16
_NW = _NC * _NS
_BPW = _BT // _NW
_CH = 128
_NCH = _BPW // _CH


def _gather_body(cb_hbm, idx_hbm, out_hbm, idx_v, rows_v, sem):
    wid = lax.axis_index("s") * _NC + lax.axis_index("c")
    pltpu.sync_copy(idx_hbm.at[pl.ds(wid * _NCH, _NCH)], idx_v)
    copies = [
        pltpu.async_copy(cb_hbm.at[idx_v.at[j]],
                         rows_v.at[pl.ds(j * _CH, _CH)], sem)
        for j in range(_NCH)
    ]
    for cp in copies:
        cp.wait()
    pltpu.sync_copy(rows_v, out_hbm.at[pl.ds(wid * _BPW, _BPW)])


def _sc_gather(cb, idx2d):
    mesh = plsc.VectorSubcoreMesh(core_axis_name="c", subcore_axis_name="s")
    run = pl.kernel(
        _gather_body,
        out_type=jax.ShapeDtypeStruct((_BT, _N_OUT), jnp.float32),
        mesh=mesh,
        scratch_types=[
            pltpu.VMEM((_NCH, _CH), jnp.int32),
            pltpu.VMEM((_BPW, _N_OUT), jnp.float32),
            pltpu.SemaphoreType.DMA,
        ],
        compiler_params=pltpu.CompilerParams(use_tc_tiling_on_sc=False),
    )
    return run(cb, idx2d)


def kernel(x, W, b, codebook):
    x2d = x.reshape(_BT, _N_IN)
    idx = _nearest_idx(x2d, W, b.reshape(1, _N_OUT), codebook)
    quant = _sc_gather(codebook, idx.reshape(_BT // _CH, _CH))
    return quant.reshape(_B, _T, _N_OUT)


# speedup vs baseline: 1.3187x; 1.3187x over previous
import functools

import jax
import jax.numpy as jnp
from jax import lax
from jax.experimental import pallas as pl
from jax.experimental.pallas import tpu as pltpu
from jax.experimental.pallas import tpu_sc as plsc

_B, _T, _N_IN, _N_OUT, _K = 8, 1024, 96, 32, 8192
_BT = _B * _T

_TM = 256
_KC = 2048


def _argmin_body(x_ref, w_ref, b_ref, cb_ref, idx_ref):
    xt = x_ref[...]
    wt = w_ref[...]
    out = lax.dot_general(xt, wt, (((1,), (1,)), ((), ())),
                          preferred_element_type=jnp.float32)
    out = out + b_ref[0, :][None, :]
    out2 = jnp.sum(out * out, axis=1, keepdims=True)
    cb = cb_ref[...]
    c2 = jnp.sum(cb * cb, axis=1)[None, :]
    s = lax.dot_general(-2.0 * out, cb, (((1,), (1,)), ((), ())),
                        preferred_element_type=jnp.float32)
    d = (out2 + s) + c2
    idx_ref[...] = jnp.argmin(d, axis=1).astype(jnp.int32).reshape(1, 1, _TM)


def _nearest_idx(x2d, w, b2d, cb):
    grid = (_BT // _TM,)
    return pl.pallas_call(
        _argmin_body,
        grid=grid,
        in_specs=[
            pl.BlockSpec((_TM, _N_IN), lambda i: (i, 0)),
            pl.BlockSpec((_N_OUT, _N_IN), lambda i: (0, 0)),
            pl.BlockSpec((1, _N_OUT), lambda i: (0, 0)),
            pl.BlockSpec((_K, _N_OUT), lambda i: (0, 0)),
        ],
        out_specs=pl.BlockSpec((1, 1, _TM), lambda i: (i, 0, 0)),
        out_shape=jax.ShapeDtypeStruct((_BT // _TM, 1, _TM), jnp.int32),
    )(x2d, w, b2d, cb)


_NC, _NS = 2, 16
_NW = _NC * _NS
_BPW = _BT // _NW
_CH = 128
_NCH = _BPW // _CH


def _gather_body(cb_hbm, idx_hbm, out_hbm, idx_v, rows_v, sem):
    wid = lax.axis_index("s") * _NC + lax.axis_index("c")
    pltpu.sync_copy(idx_hbm.at[pl.ds(wid * _NCH, _NCH)], idx_v)
    copies = [
        pltpu.async_copy(cb_hbm.at[idx_v.at[j]],
                         rows_v.at[pl.ds(j * _CH, _CH)], sem)
        for j in range(_NCH)
    ]
    for cp in copies:
        cp.wait()
    pltpu.sync_copy(rows_v, out_hbm.at[pl.ds(wid * _BPW, _BPW)])


def _sc_gather(cb, idx2d):
    mesh = plsc.VectorSubcoreMesh(core_axis_name="c", subcore_axis_name="s")
    run = pl.kernel(
        _gather_body,
        out_type=jax.ShapeDtypeStruct((_BT, _N_OUT), jnp.float32),
        mesh=mesh,
        scratch_types=[
            pltpu.VMEM((_NCH, _CH), jnp.int32),
            pltpu.VMEM((_BPW, _N_OUT), jnp.float32),
            pltpu.SemaphoreType.DMA,
        ],
        compiler_params=pltpu.CompilerParams(use_tc_tiling_on_sc=False),
    )
    return run(cb, idx2d)


def kernel(x, W, b, codebook):
    x2d = x.reshape(_BT, _N_IN)
    idx = _nearest_idx(x2d, W, b.reshape(1, _N_OUT), codebook)
    quant = _sc_gather(codebook, idx.reshape(_BT // _CH, _CH))
    return quant.reshape(_B, _T, _N_OUT)


# speedup vs baseline: 1.5081x; 1.1436x over previous
import functools

import jax
import jax.numpy as jnp
from jax import lax
from jax.experimental import pallas as pl
from jax.experimental.pallas import tpu as pltpu
from jax.experimental.pallas import tpu_sc as plsc

_B, _T, _N_IN, _N_OUT, _K = 8, 1024, 96, 32, 8192
_BT = _B * _T

_TM = 512
_KC = 2048


def _argmin_body(x_ref, w_ref, b_ref, cb_ref, idx_ref):
    xt = x_ref[...]
    wt = w_ref[...]
    out = lax.dot_general(xt, wt, (((1,), (1,)), ((), ())),
                          preferred_element_type=jnp.float32)
    out = out + b_ref[0, :][None, :]
    out2 = jnp.sum(out * out, axis=1, keepdims=True)
    cb = cb_ref[...]
    c2 = jnp.sum(cb * cb, axis=1)[None, :]
    s = lax.dot_general(-2.0 * out, cb, (((1,), (1,)), ((), ())),
                        preferred_element_type=jnp.float32)
    d = (out2 + s) + c2
    idx_ref[...] = jnp.argmin(d, axis=1).astype(jnp.int32).reshape(1, 1, _TM)


def _nearest_idx(x2d, w, b2d, cb):
    grid = (_BT // _TM,)
    return pl.pallas_call(
        _argmin_body,
        grid=grid,
        in_specs=[
            pl.BlockSpec((_TM, _N_IN), lambda i: (i, 0)),
            pl.BlockSpec((_N_OUT, _N_IN), lambda i: (0, 0)),
            pl.BlockSpec((1, _N_OUT), lambda i: (0, 0)),
            pl.BlockSpec((_K, _N_OUT), lambda i: (0, 0)),
        ],
        out_specs=pl.BlockSpec((1, 1, _TM), lambda i: (i, 0, 0)),
        out_shape=jax.ShapeDtypeStruct((_BT // _TM, 1, _TM), jnp.int32),
    )(x2d, w, b2d, cb)


_NC, _NS = 2, 16
_NW = _NC * _NS
_BPW = _BT // _NW
_CH = 128
_NCH = _BPW // _CH


def _gather_body(cb_hbm, idx_hbm, out_hbm, idx_v, rows_v, sem):
    wid = lax.axis_index("s") * _NC + lax.axis_index("c")
    pltpu.sync_copy(idx_hbm.at[pl.ds(wid * _NCH, _NCH)], idx_v)
    copies = [
        pltpu.async_copy(cb_hbm.at[idx_v.at[j]],
                         rows_v.at[pl.ds(j * _CH, _CH)], sem)
        for j in range(_NCH)
    ]
    for cp in copies:
        cp.wait()
    pltpu.sync_copy(rows_v, out_hbm.at[pl.ds(wid * _BPW, _BPW)])


def _sc_gather(cb, idx2d):
    mesh = plsc.VectorSubcoreMesh(core_axis_name="c", subcore_axis_name="s")
    run = pl.kernel(
        _gather_body,
        out_type=jax.ShapeDtypeStruct((_BT, _N_OUT), jnp.float32),
        mesh=mesh,
        scratch_types=[
            pltpu.VMEM((_NCH, _CH), jnp.int32),
            pltpu.VMEM((_BPW, _N_OUT), jnp.float32),
            pltpu.SemaphoreType.DMA,
        ],
        compiler_params=pltpu.CompilerParams(use_tc_tiling_on_sc=False),
    )
    return run(cb, idx2d)


def kernel(x, W, b, codebook):
    x2d = x.reshape(_BT, _N_IN)
    idx = _nearest_idx(x2d, W, b.reshape(1, _N_OUT), codebook)
    quant = _sc_gather(codebook, idx.reshape(_BT // _CH, _CH))
    return quant.reshape(_B, _T, _N_OUT)


# speedup vs baseline: 1.5980x; 1.0596x over previous
import functools

import jax
import jax.numpy as jnp
from jax import lax
from jax.experimental import pallas as pl
from jax.experimental.pallas import tpu as pltpu
from jax.experimental.pallas import tpu_sc as plsc

_B, _T, _N_IN, _N_OUT, _K = 8, 1024, 96, 32, 8192
_BT = _B * _T

_TM = 1024
_KC = 2048


def _argmin_body(x_ref, w_ref, b_ref, cb_ref, idx_ref):
    xt = x_ref[...]
    wt = w_ref[...]
    out = lax.dot_general(xt, wt, (((1,), (1,)), ((), ())),
                          preferred_element_type=jnp.float32)
    out = out + b_ref[0, :][None, :]
    out2 = jnp.sum(out * out, axis=1, keepdims=True)
    cb = cb_ref[...]
    c2 = jnp.sum(cb * cb, axis=1)[None, :]
    s = lax.dot_general(-2.0 * out, cb, (((1,), (1,)), ((), ())),
                        preferred_element_type=jnp.float32)
    d = (out2 + s) + c2
    idx_ref[...] = jnp.argmin(d, axis=1).astype(jnp.int32).reshape(1, 1, _TM)


def _nearest_idx(x2d, w, b2d, cb):
    grid = (_BT // _TM,)
    return pl.pallas_call(
        _argmin_body,
        grid=grid,
        in_specs=[
            pl.BlockSpec((_TM, _N_IN), lambda i: (i, 0)),
            pl.BlockSpec((_N_OUT, _N_IN), lambda i: (0, 0)),
            pl.BlockSpec((1, _N_OUT), lambda i: (0, 0)),
            pl.BlockSpec((_K, _N_OUT), lambda i: (0, 0)),
        ],
        out_specs=pl.BlockSpec((1, 1, _TM), lambda i: (i, 0, 0)),
        out_shape=jax.ShapeDtypeStruct((_BT // _TM, 1, _TM), jnp.int32),
    )(x2d, w, b2d, cb)


_NC, _NS = 2, 16
_NW = _NC * _NS
_BPW = _BT // _NW
_CH = 128
_NCH = _BPW // _CH


def _gather_body(cb_hbm, idx_hbm, out_hbm, idx_v, rows_v, sem):
    wid = lax.axis_index("s") * _NC + lax.axis_index("c")
    pltpu.sync_copy(idx_hbm.at[pl.ds(wid * _NCH, _NCH)], idx_v)
    copies = [
        pltpu.async_copy(cb_hbm.at[idx_v.at[j]],
                         rows_v.at[pl.ds(j * _CH, _CH)], sem)
        for j in range(_NCH)
    ]
    for cp in copies:
        cp.wait()
    pltpu.sync_copy(rows_v, out_hbm.at[pl.ds(wid * _BPW, _BPW)])


def _sc_gather(cb, idx2d):
    mesh = plsc.VectorSubcoreMesh(core_axis_name="c", subcore_axis_name="s")
    run = pl.kernel(
        _gather_body,
        out_type=jax.ShapeDtypeStruct((_BT, _N_OUT), jnp.float32),
        mesh=mesh,
        scratch_types=[
            pltpu.VMEM((_NCH, _CH), jnp.int32),
            pltpu.VMEM((_BPW, _N_OUT), jnp.float32),
            pltpu.SemaphoreType.DMA,
        ],
        compiler_params=pltpu.CompilerParams(use_tc_tiling_on_sc=False),
    )
    return run(cb, idx2d)


def kernel(x, W, b, codebook):
    x2d = x.reshape(_BT, _N_IN)
    idx = _nearest_idx(x2d, W, b.reshape(1, _N_OUT), codebook)
    quant = _sc_gather(codebook, idx.reshape(_BT // _CH, _CH))
    return quant.reshape(_B, _T, _N_OUT)
